# baseline (speedup 1.0000x reference)
import functools

import jax
import jax.numpy as jnp
from jax import lax
from jax.experimental import pallas as pl
from jax.experimental.pallas import tpu as pltpu
from jax.experimental.pallas import tpu_sc as plsc

N_ATOMS = 10000
MAX_NEI = 32
HIDDEN = 128
ATOM_FDIM = 133
BOND_FDIM = 14
DEPTH = 3

NW = 32
APW = 320
N_PAD = NW * APW
CHUNK_ATOMS = 4
CHUNKS = APW // CHUNK_ATOMS


def _gelu_exact(x):
    return 0.5 * x * (1.0 + lax.erf(x * 0.7071067811865476))


_MGROUPS = HIDDEN // 32
_FLUSH = 8


def _reduce_msg_chunk(rows_v, out_v, b):
    for a in range(CHUNK_ATOMS):
        f32accs = [None] * (2 * _MGROUPS)
        for r0 in range(0, MAX_NEI, _FLUSH):
            bacc = [rows_v[b, a * MAX_NEI + r0, pl.ds(32 * g, 32)]
                    for g in range(_MGROUPS)]
            for r in range(r0 + 1, r0 + _FLUSH):
                for g in range(_MGROUPS):
                    bacc[g] = bacc[g] + rows_v[
                        b, a * MAX_NEI + r, pl.ds(32 * g, 32)]
            for g in range(_MGROUPS):
                lo, hi = plsc.unpack(bacc[g],
                                     format=plsc.PackFormat.INTERLEAVED)
                if f32accs[2 * g] is None:
                    f32accs[2 * g], f32accs[2 * g + 1] = lo, hi
                else:
                    f32accs[2 * g] = f32accs[2 * g] + lo
                    f32accs[2 * g + 1] = f32accs[2 * g + 1] + hi
        for g in range(_MGROUPS):
            out_v[b, a, pl.ds(32 * g, 32)] = plsc.pack(
                f32accs[2 * g], f32accs[2 * g + 1],
                format=plsc.PackFormat.INTERLEAVED)


def _reduce_bond_chunk(rows_v, out_v, b):
    for a in range(CHUNK_ATOMS):
        acc = rows_v[b, a * MAX_NEI, pl.ds(0, 16)]
        for r in range(1, MAX_NEI):
            acc = acc + rows_v[b, a * MAX_NEI + r, pl.ds(0, 16)]
        out_v[b, a, pl.ds(0, 16)] = acc


def _make_sc_round(with_bond):
    mesh = plsc.VectorSubcoreMesh(core_axis_name="c", subcore_axis_name="s")
    nbuf = 4
    out_type = [jax.ShapeDtypeStruct((N_PAD, HIDDEN), jnp.bfloat16)]
    scratch = [
        pltpu.VMEM((CHUNKS, 128), jnp.int32),
        pltpu.VMEM((nbuf, 128, HIDDEN), jnp.bfloat16),
        pltpu.VMEM((nbuf, CHUNK_ATOMS, HIDDEN), jnp.bfloat16),
        pltpu.VMEM_SHARED((N_ATOMS, HIDDEN), jnp.bfloat16),
    ] + [pltpu.SemaphoreType.DMA] * (2 * nbuf)
    if with_bond:
        out_type.append(jax.ShapeDtypeStruct((N_PAD, 16), jnp.float32))
        scratch += [
            pltpu.VMEM((CHUNKS, 128), jnp.int32),
            pltpu.VMEM((nbuf, 128, 16), jnp.float32),
            pltpu.VMEM((nbuf, CHUNK_ATOMS, 16), jnp.float32),
        ] + [pltpu.SemaphoreType.DMA] * (2 * nbuf)

    def body(refs):
        if with_bond:
            (mtab, idxa_hbm, btab, idxb_hbm, s_out, b_out,
             idxa_v, rows_m, outm_v, shared, *rest) = refs
            msems, mosems = rest[:nbuf], rest[nbuf:2 * nbuf]
            idxb_v, rows_b, outb_v = rest[2 * nbuf:2 * nbuf + 3]
            bsems = rest[2 * nbuf + 3:3 * nbuf + 3]
            bosems = rest[3 * nbuf + 3:]
        else:
            (mtab, idxa_hbm, s_out,
             idxa_v, rows_m, outm_v, shared, *rest) = refs
            msems, mosems = rest[:nbuf], rest[nbuf:2 * nbuf]

        wid = lax.axis_index("s") * 2 + lax.axis_index("c")
        pltpu.sync_copy(idxa_hbm.at[wid], idxa_v)
        if with_bond:
            pltpu.sync_copy(idxb_hbm.at[wid], idxb_v)

        def mcopy(c, b):
            return pltpu.make_async_copy(
                shared.at[idxa_v.at[c]], rows_m.at[b], msems[b])

        def mout(c, b):
            return pltpu.make_async_copy(
                outm_v.at[b],
                s_out.at[pl.ds(wid * APW + c * CHUNK_ATOMS, CHUNK_ATOMS)],
                mosems[b])

        if with_bond:
            def bcopy(c, b):
                return pltpu.make_async_copy(
                    btab.at[idxb_v.at[c]], rows_b.at[b], bsems[b])

            def bout(c, b):
                return pltpu.make_async_copy(
                    outb_v.at[b],
                    b_out.at[pl.ds(wid * APW + c * CHUNK_ATOMS, CHUNK_ATOMS)],
                    bosems[b])

        sub = lax.axis_index("s")
        rpw = N_ATOMS // 16
        if with_bond:
            for b in range(nbuf - 1):
                bcopy(b, b).start()
        pltpu.sync_copy(mtab.at[pl.ds(sub * rpw, rpw)],
                        shared.at[pl.ds(sub * rpw, rpw)])
        plsc.subcore_barrier()

        for b in range(nbuf - 1):
            mcopy(b, b).start()

        def quad_body(i, _):
            c0 = i * nbuf
            for b in range(nbuf):
                c = c0 + b

                @pl.when(c + nbuf - 1 < CHUNKS)
                def _():
                    mcopy(c + nbuf - 1, (b + nbuf - 1) % nbuf).start()
                    if with_bond:
                        bcopy(c + nbuf - 1, (b + nbuf - 1) % nbuf).start()

                mcopy(c, b).wait()

                @pl.when(c >= nbuf)
                def _():
                    mout(c - nbuf, b).wait()
                    if with_bond:
                        bout(c - nbuf, b).wait()

                _reduce_msg_chunk(rows_m, outm_v, b)
                mout(c, b).start()
                if with_bond:
                    bcopy(c, b).wait()
                    _reduce_bond_chunk(rows_b, outb_v, b)
                    bout(c, b).start()
            return 0

        lax.fori_loop(0, CHUNKS // nbuf, quad_body, 0)
        for b in range(nbuf):
            mout(CHUNKS - nbuf + b, b).wait()
            if with_bond:
                bout(CHUNKS - nbuf + b, b).wait()

    def entry(*refs):
        body(refs)

    return functools.partial(
        pl.kernel,
        out_type=out_type if with_bond else out_type[0],
        mesh=mesh,
        compiler_params=pltpu.CompilerParams(use_tc_tiling_on_sc=False,
                                             needs_layout_passes=False),
        scratch_types=scratch,
    )(entry)


_round_fused = _make_sc_round(True)
_round_msg = _make_sc_round(False)


_ROWS = 2000
_GRID = N_ATOMS // _ROWS


def _row_mask(pid, x):
    rows = lax.broadcasted_iota(jnp.int32, x.shape, 0) + pid * _ROWS
    return jnp.where(rows == 0, 0.0, x)


def _prologue_body(x_ref, wi_ref, w0_ref, w1_ref, w2_ref,
                   inp_ref, inpb_ref, h_ref):
    pid = pl.program_id(0)
    x = x_ref[...]
    inp = jnp.dot(x, wi_ref[...], preferred_element_type=jnp.float32)
    inp = _row_mask(pid, inp)
    inp_ref[...] = inp
    inpb_ref[...] = inp.astype(jnp.bfloat16)
    h = _gelu_exact(jnp.dot(x, w0_ref[...], preferred_element_type=jnp.float32))
    h = _gelu_exact(jnp.dot(h, w1_ref[...], preferred_element_type=jnp.float32))
    h = _gelu_exact(jnp.dot(h, w2_ref[...], preferred_element_type=jnp.float32))
    h_ref[...] = h


def _tc_prologue(f_atoms, W_i, W0, W1, W2):
    return pl.pallas_call(
        _prologue_body,
        grid=(_GRID,),
        in_specs=[
            pl.BlockSpec((_ROWS, ATOM_FDIM), lambda i: (i, 0)),
            pl.BlockSpec((ATOM_FDIM, HIDDEN), lambda i: (0, 0)),
            pl.BlockSpec((ATOM_FDIM, HIDDEN), lambda i: (0, 0)),
            pl.BlockSpec((HIDDEN, HIDDEN), lambda i: (0, 0)),
            pl.BlockSpec((HIDDEN, HIDDEN), lambda i: (0, 0)),
        ],
        out_specs=[
            pl.BlockSpec((_ROWS, HIDDEN), lambda i: (i, 0)),
            pl.BlockSpec((_ROWS, HIDDEN), lambda i: (i, 0)),
            pl.BlockSpec((_ROWS, HIDDEN), lambda i: (i, 0)),
        ],
        out_shape=[
            jax.ShapeDtypeStruct((N_ATOMS, HIDDEN), jnp.float32),
            jax.ShapeDtypeStruct((N_ATOMS, HIDDEN), jnp.bfloat16),
            jax.ShapeDtypeStruct((N_ATOMS, HIDDEN), jnp.float32),
        ],
    )(f_atoms, W_i, W0, W1, W2)


def _update_body(m_ref, s_ref, b_ref, wt_ref, wb_ref, o_ref, ob_ref):
    pid = pl.program_id(0)
    s = s_ref[...].astype(jnp.float32)
    m = (m_ref[...]
         + jnp.dot(s, wt_ref[...], preferred_element_type=jnp.float32)
         + jnp.dot(b_ref[...], wb_ref[...], preferred_element_type=jnp.float32))
    m = _row_mask(pid, m)
    o_ref[...] = m
    ob_ref[...] = m.astype(jnp.bfloat16)


def _tc_update(message, s, sumb, Wh_top, Wh_bot16):
    return pl.pallas_call(
        _update_body,
        grid=(_GRID,),
        in_specs=[
            pl.BlockSpec((_ROWS, HIDDEN), lambda i: (i, 0)),
            pl.BlockSpec((_ROWS, HIDDEN), lambda i: (i, 0)),
            pl.BlockSpec((_ROWS, 16), lambda i: (i, 0)),
            pl.BlockSpec((HIDDEN, HIDDEN), lambda i: (0, 0)),
            pl.BlockSpec((16, HIDDEN), lambda i: (0, 0)),
        ],
        out_specs=[
            pl.BlockSpec((_ROWS, HIDDEN), lambda i: (i, 0)),
            pl.BlockSpec((_ROWS, HIDDEN), lambda i: (i, 0)),
        ],
        out_shape=[
            jax.ShapeDtypeStruct((N_ATOMS, HIDDEN), jnp.float32),
            jax.ShapeDtypeStruct((N_ATOMS, HIDDEN), jnp.bfloat16),
        ],
    )(message, s, sumb, Wh_top, Wh_bot16)


def _final_body(h_ref, s_ref, wt_ref, wb_ref, o_ref):
    s = s_ref[...].astype(jnp.float32)
    o = (jnp.dot(h_ref[...], wt_ref[...], preferred_element_type=jnp.float32)
         + jnp.dot(s, wb_ref[...], preferred_element_type=jnp.float32))
    o_ref[...] = _gelu_exact(o)


def _tc_final(h, s, Wo_top, Wo_bot):
    return pl.pallas_call(
        _final_body,
        grid=(_GRID,),
        in_specs=[
            pl.BlockSpec((_ROWS, HIDDEN), lambda i: (i, 0)),
            pl.BlockSpec((_ROWS, HIDDEN), lambda i: (i, 0)),
            pl.BlockSpec((HIDDEN, HIDDEN), lambda i: (0, 0)),
            pl.BlockSpec((HIDDEN, HIDDEN), lambda i: (0, 0)),
        ],
        out_specs=pl.BlockSpec((_ROWS, HIDDEN), lambda i: (i, 0)),
        out_shape=jax.ShapeDtypeStruct((N_ATOMS, HIDDEN), jnp.float32),
    )(h, s, Wo_top, Wo_bot)


def _pack_idx(idx):
    idx = jnp.pad(idx.astype(jnp.int32), ((0, N_PAD - N_ATOMS), (0, 0)))
    return idx.reshape(NW, CHUNKS, 128)


def kernel(f_atoms, f_bonds, a2a, a2b, W_i, W_ah0, W_ah1, W_ah2,
           W_h0, W_h1, W_h2, W_o):
    idx_a = _pack_idx(a2a)
    idx_b = _pack_idx(a2b)
    f_bonds16 = jnp.pad(f_bonds, ((0, 0), (0, 16 - BOND_FDIM)))

    W_h = [W_h0, W_h1, W_h2]
    Wh_top = [w[:HIDDEN] for w in W_h]
    Wh_bot16 = [jnp.pad(w[HIDDEN:], ((0, 2), (0, 0))) for w in W_h]

    inp, inp_bf, h = _tc_prologue(f_atoms, W_i, W_ah0, W_ah1, W_ah2)

    message, message_bf = inp, inp_bf
    sumb = None
    for d in range(DEPTH):
        if d == 0:
            s, sumb = _round_fused(message_bf, idx_a, f_bonds16, idx_b)
            s, sumb = s[:N_ATOMS], sumb[:N_ATOMS]
        else:
            s = _round_msg(message_bf, idx_a)[:N_ATOMS]
        message, message_bf = _tc_update(message, s, sumb,
                                         Wh_top[d], Wh_bot16[d])

    s = _round_msg(message_bf, idx_a)[:N_ATOMS]
    return _tc_final(h, s, W_o[:HIDDEN], W_o[HIDDEN:])


# speedup vs baseline: 1.0511x; 1.0511x over previous
import functools

import jax
import jax.numpy as jnp
from jax import lax
from jax.experimental import pallas as pl
from jax.experimental.pallas import tpu as pltpu
from jax.experimental.pallas import tpu_sc as plsc

N_ATOMS = 10000
MAX_NEI = 32
HIDDEN = 128
ATOM_FDIM = 133
BOND_FDIM = 14
DEPTH = 3

NW = 32
APW = 320
N_PAD = NW * APW
CHUNK_ATOMS = 4
CHUNKS = APW // CHUNK_ATOMS


def _gelu_exact(x):
    return 0.5 * x * (1.0 + lax.erf(x * 0.7071067811865476))


_MGROUPS = HIDDEN // 32
_FLUSH = 8


def _reduce_msg_chunk(rows_v, out_v, b):
    for a in range(CHUNK_ATOMS):
        f32accs = [None] * (2 * _MGROUPS)
        for r in range(MAX_NEI):
            for g in range(_MGROUPS):
                lo, hi = plsc.unpack(
                    rows_v[b, a * MAX_NEI + r, pl.ds(32 * g, 32)],
                    format=plsc.PackFormat.INTERLEAVED)
                if f32accs[2 * g] is None:
                    f32accs[2 * g], f32accs[2 * g + 1] = lo, hi
                else:
                    f32accs[2 * g] = f32accs[2 * g] + lo
                    f32accs[2 * g + 1] = f32accs[2 * g + 1] + hi
        for g in range(_MGROUPS):
            out_v[b, a, pl.ds(32 * g, 32)] = plsc.pack(
                f32accs[2 * g], f32accs[2 * g + 1],
                format=plsc.PackFormat.INTERLEAVED)


def _reduce_bond_chunk(rows_v, out_v, b):
    for a in range(CHUNK_ATOMS):
        acc = rows_v[b, a * MAX_NEI, pl.ds(0, 16)]
        for r in range(1, MAX_NEI):
            acc = acc + rows_v[b, a * MAX_NEI + r, pl.ds(0, 16)]
        out_v[b, a, pl.ds(0, 16)] = acc


def _make_sc_round(with_bond):
    mesh = plsc.VectorSubcoreMesh(core_axis_name="c", subcore_axis_name="s")
    nbuf = 4
    out_type = [jax.ShapeDtypeStruct((N_PAD, HIDDEN), jnp.bfloat16)]
    scratch = [
        pltpu.VMEM((CHUNKS, 128), jnp.int32),
        pltpu.VMEM((nbuf, 128, HIDDEN), jnp.bfloat16),
        pltpu.VMEM((nbuf, CHUNK_ATOMS, HIDDEN), jnp.bfloat16),
        pltpu.VMEM_SHARED((N_ATOMS, HIDDEN), jnp.bfloat16),
    ] + [pltpu.SemaphoreType.DMA] * (2 * nbuf)
    if with_bond:
        out_type.append(jax.ShapeDtypeStruct((N_PAD, 16), jnp.float32))
        scratch += [
            pltpu.VMEM((CHUNKS, 128), jnp.int32),
            pltpu.VMEM((nbuf, 128, 16), jnp.float32),
            pltpu.VMEM((nbuf, CHUNK_ATOMS, 16), jnp.float32),
        ] + [pltpu.SemaphoreType.DMA] * (2 * nbuf)

    def body(refs):
        if with_bond:
            (mtab, idxa_hbm, btab, idxb_hbm, s_out, b_out,
             idxa_v, rows_m, outm_v, shared, *rest) = refs
            msems, mosems = rest[:nbuf], rest[nbuf:2 * nbuf]
            idxb_v, rows_b, outb_v = rest[2 * nbuf:2 * nbuf + 3]
            bsems = rest[2 * nbuf + 3:3 * nbuf + 3]
            bosems = rest[3 * nbuf + 3:]
        else:
            (mtab, idxa_hbm, s_out,
             idxa_v, rows_m, outm_v, shared, *rest) = refs
            msems, mosems = rest[:nbuf], rest[nbuf:2 * nbuf]

        wid = lax.axis_index("s") * 2 + lax.axis_index("c")
        pltpu.sync_copy(idxa_hbm.at[wid], idxa_v)
        if with_bond:
            pltpu.sync_copy(idxb_hbm.at[wid], idxb_v)

        def mcopy(c, b):
            return pltpu.make_async_copy(
                shared.at[idxa_v.at[c]], rows_m.at[b], msems[b])

        def mout(c, b):
            return pltpu.make_async_copy(
                outm_v.at[b],
                s_out.at[pl.ds(wid * APW + c * CHUNK_ATOMS, CHUNK_ATOMS)],
                mosems[b])

        if with_bond:
            def bcopy(c, b):
                return pltpu.make_async_copy(
                    btab.at[idxb_v.at[c]], rows_b.at[b], bsems[b])

            def bout(c, b):
                return pltpu.make_async_copy(
                    outb_v.at[b],
                    b_out.at[pl.ds(wid * APW + c * CHUNK_ATOMS, CHUNK_ATOMS)],
                    bosems[b])

        sub = lax.axis_index("s")
        rpw = N_ATOMS // 16
        if with_bond:
            for b in range(nbuf - 1):
                bcopy(b, b).start()
        pltpu.sync_copy(mtab.at[pl.ds(sub * rpw, rpw)],
                        shared.at[pl.ds(sub * rpw, rpw)])
        plsc.subcore_barrier()

        for b in range(nbuf - 1):
            mcopy(b, b).start()

        def quad_body(i, _):
            c0 = i * nbuf
            for b in range(nbuf):
                c = c0 + b

                @pl.when(c + nbuf - 1 < CHUNKS)
                def _():
                    mcopy(c + nbuf - 1, (b + nbuf - 1) % nbuf).start()
                    if with_bond:
                        bcopy(c + nbuf - 1, (b + nbuf - 1) % nbuf).start()

                mcopy(c, b).wait()

                @pl.when(c >= nbuf)
                def _():
                    mout(c - nbuf, b).wait()
                    if with_bond:
                        bout(c - nbuf, b).wait()

                _reduce_msg_chunk(rows_m, outm_v, b)
                mout(c, b).start()
                if with_bond:
                    bcopy(c, b).wait()
                    _reduce_bond_chunk(rows_b, outb_v, b)
                    bout(c, b).start()
            return 0

        lax.fori_loop(0, CHUNKS // nbuf, quad_body, 0)
        for b in range(nbuf):
            mout(CHUNKS - nbuf + b, b).wait()
            if with_bond:
                bout(CHUNKS - nbuf + b, b).wait()

    def entry(*refs):
        body(refs)

    return functools.partial(
        pl.kernel,
        out_type=out_type if with_bond else out_type[0],
        mesh=mesh,
        compiler_params=pltpu.CompilerParams(use_tc_tiling_on_sc=False,
                                             needs_layout_passes=False),
        scratch_types=scratch,
    )(entry)


_round_fused = _make_sc_round(True)
_round_msg = _make_sc_round(False)


_ROWS = 2000
_GRID = N_ATOMS // _ROWS


def _row_mask(pid, x):
    rows = lax.broadcasted_iota(jnp.int32, x.shape, 0) + pid * _ROWS
    return jnp.where(rows == 0, 0.0, x)


def _prologue_body(x_ref, wi_ref, w0_ref, w1_ref, w2_ref,
                   inp_ref, inpb_ref, h_ref):
    pid = pl.program_id(0)
    x = x_ref[...]
    inp = jnp.dot(x, wi_ref[...], preferred_element_type=jnp.float32)
    inp = _row_mask(pid, inp)
    inp_ref[...] = inp
    inpb_ref[...] = inp.astype(jnp.bfloat16)
    h = _gelu_exact(jnp.dot(x, w0_ref[...], preferred_element_type=jnp.float32))
    h = _gelu_exact(jnp.dot(h, w1_ref[...], preferred_element_type=jnp.float32))
    h = _gelu_exact(jnp.dot(h, w2_ref[...], preferred_element_type=jnp.float32))
    h_ref[...] = h


def _tc_prologue(f_atoms, W_i, W0, W1, W2):
    return pl.pallas_call(
        _prologue_body,
        grid=(_GRID,),
        in_specs=[
            pl.BlockSpec((_ROWS, ATOM_FDIM), lambda i: (i, 0)),
            pl.BlockSpec((ATOM_FDIM, HIDDEN), lambda i: (0, 0)),
            pl.BlockSpec((ATOM_FDIM, HIDDEN), lambda i: (0, 0)),
            pl.BlockSpec((HIDDEN, HIDDEN), lambda i: (0, 0)),
            pl.BlockSpec((HIDDEN, HIDDEN), lambda i: (0, 0)),
        ],
        out_specs=[
            pl.BlockSpec((_ROWS, HIDDEN), lambda i: (i, 0)),
            pl.BlockSpec((_ROWS, HIDDEN), lambda i: (i, 0)),
            pl.BlockSpec((_ROWS, HIDDEN), lambda i: (i, 0)),
        ],
        out_shape=[
            jax.ShapeDtypeStruct((N_ATOMS, HIDDEN), jnp.float32),
            jax.ShapeDtypeStruct((N_ATOMS, HIDDEN), jnp.bfloat16),
            jax.ShapeDtypeStruct((N_ATOMS, HIDDEN), jnp.float32),
        ],
    )(f_atoms, W_i, W0, W1, W2)


def _update_body(m_ref, s_ref, b_ref, wt_ref, wb_ref, o_ref, ob_ref):
    pid = pl.program_id(0)
    s = s_ref[...].astype(jnp.float32)
    m = (m_ref[...]
         + jnp.dot(s, wt_ref[...], preferred_element_type=jnp.float32)
         + jnp.dot(b_ref[...], wb_ref[...], preferred_element_type=jnp.float32))
    m = _row_mask(pid, m)
    o_ref[...] = m
    ob_ref[...] = m.astype(jnp.bfloat16)


def _tc_update(message, s, sumb, Wh_top, Wh_bot16):
    return pl.pallas_call(
        _update_body,
        grid=(_GRID,),
        in_specs=[
            pl.BlockSpec((_ROWS, HIDDEN), lambda i: (i, 0)),
            pl.BlockSpec((_ROWS, HIDDEN), lambda i: (i, 0)),
            pl.BlockSpec((_ROWS, 16), lambda i: (i, 0)),
            pl.BlockSpec((HIDDEN, HIDDEN), lambda i: (0, 0)),
            pl.BlockSpec((16, HIDDEN), lambda i: (0, 0)),
        ],
        out_specs=[
            pl.BlockSpec((_ROWS, HIDDEN), lambda i: (i, 0)),
            pl.BlockSpec((_ROWS, HIDDEN), lambda i: (i, 0)),
        ],
        out_shape=[
            jax.ShapeDtypeStruct((N_ATOMS, HIDDEN), jnp.float32),
            jax.ShapeDtypeStruct((N_ATOMS, HIDDEN), jnp.bfloat16),
        ],
    )(message, s, sumb, Wh_top, Wh_bot16)


def _final_body(h_ref, s_ref, wt_ref, wb_ref, o_ref):
    s = s_ref[...].astype(jnp.float32)
    o = (jnp.dot(h_ref[...], wt_ref[...], preferred_element_type=jnp.float32)
         + jnp.dot(s, wb_ref[...], preferred_element_type=jnp.float32))
    o_ref[...] = _gelu_exact(o)


def _tc_final(h, s, Wo_top, Wo_bot):
    return pl.pallas_call(
        _final_body,
        grid=(_GRID,),
        in_specs=[
            pl.BlockSpec((_ROWS, HIDDEN), lambda i: (i, 0)),
            pl.BlockSpec((_ROWS, HIDDEN), lambda i: (i, 0)),
            pl.BlockSpec((HIDDEN, HIDDEN), lambda i: (0, 0)),
            pl.BlockSpec((HIDDEN, HIDDEN), lambda i: (0, 0)),
        ],
        out_specs=pl.BlockSpec((_ROWS, HIDDEN), lambda i: (i, 0)),
        out_shape=jax.ShapeDtypeStruct((N_ATOMS, HIDDEN), jnp.float32),
    )(h, s, Wo_top, Wo_bot)


def _pack_idx(idx):
    idx = jnp.pad(idx.astype(jnp.int32), ((0, N_PAD - N_ATOMS), (0, 0)))
    return idx.reshape(NW, CHUNKS, 128)


def kernel(f_atoms, f_bonds, a2a, a2b, W_i, W_ah0, W_ah1, W_ah2,
           W_h0, W_h1, W_h2, W_o):
    idx_a = _pack_idx(a2a)
    idx_b = _pack_idx(a2b)
    f_bonds16 = jnp.pad(f_bonds, ((0, 0), (0, 16 - BOND_FDIM)))

    W_h = [W_h0, W_h1, W_h2]
    Wh_top = [w[:HIDDEN] for w in W_h]
    Wh_bot16 = [jnp.pad(w[HIDDEN:], ((0, 2), (0, 0))) for w in W_h]

    inp, inp_bf, h = _tc_prologue(f_atoms, W_i, W_ah0, W_ah1, W_ah2)

    message, message_bf = inp, inp_bf
    sumb = None
    for d in range(DEPTH):
        if d == 0:
            s, sumb = _round_fused(message_bf, idx_a, f_bonds16, idx_b)
            s, sumb = s[:N_ATOMS], sumb[:N_ATOMS]
        else:
            s = _round_msg(message_bf, idx_a)[:N_ATOMS]
        message, message_bf = _tc_update(message, s, sumb,
                                         Wh_top[d], Wh_bot16[d])

    s = _round_msg(message_bf, idx_a)[:N_ATOMS]
    return _tc_final(h, s, W_o[:HIDDEN], W_o[HIDDEN:])
